# Initial kernel scaffold; baseline (speedup 1.0000x reference)
import jax
import jax.numpy as jnp
from jax.experimental import pallas as pl


def kernel(node_feat, coord_feat, edge_feat, edge_index, We1, be1, We2, be2, Wn1, bn1, Wn2, bn2, Wc1, bc1, Wc2):
    raise NotImplementedError("write your pallas kernel here")


# speedup vs baseline: 2.9626x; 2.9626x over previous
import functools

import jax
import jax.numpy as jnp
from jax import lax
from jax.experimental import pallas as pl
from jax.experimental.pallas import tpu as pltpu
from jax.experimental.pallas import tpu_sc as plsc

_N = 10000
_E = 320000
_D = 128
_H = 128
_DE = 16
_CD = 3

_NC = 2
_NS = 16
_NW = _NC * _NS
_EPW = _E // _NW
_KB = 80
_NCHUNK = _EPW // _KB

_mesh = plsc.VectorSubcoreMesh(core_axis_name="c", subcore_axis_name="s")


def _silu(x):
    return x / (1.0 + jnp.exp(-x))


def _stage_a_body(node_ref, ws_ref, wd_ref, be1_ref, ts_ref, td_ref):
    n = node_ref[...]
    ts_ref[...] = jnp.dot(n, ws_ref[...],
                          preferred_element_type=jnp.float32) + be1_ref[...]
    td_ref[...] = jnp.dot(n, wd_ref[...], preferred_element_type=jnp.float32)


def _stage_a(node, Ws, Wd, be1):
    NB = 1000
    return pl.pallas_call(
        _stage_a_body,
        grid=(_N // NB,),
        in_specs=[
            pl.BlockSpec((NB, _D), lambda i: (i, 0)),
            pl.BlockSpec((_D, _D), lambda i: (0, 0)),
            pl.BlockSpec((_D, _D), lambda i: (0, 0)),
            pl.BlockSpec((1, _D), lambda i: (0, 0)),
        ],
        out_specs=[pl.BlockSpec((NB, _D), lambda i: (i, 0))] * 2,
        out_shape=[jax.ShapeDtypeStruct((_N, _D), jnp.float32)] * 2,
    )(node, Ws, Wd, be1)


@functools.partial(
    pl.kernel,
    mesh=_mesh,
    out_type=[jax.ShapeDtypeStruct((_E, _D), jnp.float32)] * 2
             + [jax.ShapeDtypeStruct((_E,), jnp.float32)] * 3,
    scratch_types=[
        pltpu.VMEM((_KB,), jnp.int32),
        pltpu.VMEM((_KB,), jnp.int32),
        pltpu.VMEM((_KB, _D), jnp.float32),
        pltpu.VMEM((_KB, _D), jnp.float32),
        pltpu.VMEM((6, _KB), jnp.float32),
        pltpu.VMEM((_KB,), jnp.float32),
        pltpu.VMEM((_KB,), jnp.float32),
        pltpu.VMEM((_KB,), jnp.float32),
        pltpu.SemaphoreType.DMA,
        pltpu.SemaphoreType.DMA,
    ],
)
def _stage_b(ts_hbm, td_hbm, src_hbm, dst_hbm, cx_hbm, cy_hbm, cz_hbm,
             gs_hbm, gd_hbm, xx_hbm, xy_hbm, xz_hbm,
             idxs_v, idxd_v, rs_v, rd_v, cab_v,
             bx_v, by_v, bz_v, sem_r, sem_c):
    wid = lax.axis_index("s") * _NC + lax.axis_index("c")
    base = wid * _EPW

    def body(ci, carry):
        off = base + ci * _KB
        pltpu.sync_copy(src_hbm.at[pl.ds(off, _KB)], idxs_v)
        pltpu.sync_copy(dst_hbm.at[pl.ds(off, _KB)], idxd_v)
        cps = pltpu.async_copy(ts_hbm.at[idxs_v], rs_v, sem_r)
        cpd = pltpu.async_copy(td_hbm.at[idxd_v], rd_v, sem_r)
        ccs = []
        for k, (ctab, iv) in enumerate((
                (cx_hbm, idxs_v), (cy_hbm, idxs_v), (cz_hbm, idxs_v),
                (cx_hbm, idxd_v), (cy_hbm, idxd_v), (cz_hbm, idxd_v))):
            ccs.append(pltpu.async_copy(ctab.at[iv], cab_v.at[k], sem_c))
        for cc in ccs:
            cc.wait()
        for j in range(_KB // 16):
            sl = pl.ds(j * 16, 16)
            for k, bref in ((0, bx_v), (1, by_v), (2, bz_v)):
                d = cab_v[k, sl] - cab_v[k + 3, sl]
                w = jnp.where(jnp.abs(d) > 0.5, jnp.sign(d), 0.0)
                bref[sl] = d - w
        cps.wait()
        cpd.wait()
        pltpu.sync_copy(rs_v, gs_hbm.at[pl.ds(off, _KB)])
        pltpu.sync_copy(rd_v, gd_hbm.at[pl.ds(off, _KB)])
        pltpu.sync_copy(bx_v, xx_hbm.at[pl.ds(off, _KB)])
        pltpu.sync_copy(by_v, xy_hbm.at[pl.ds(off, _KB)])
        pltpu.sync_copy(bz_v, xz_hbm.at[pl.ds(off, _KB)])
        return carry

    lax.fori_loop(0, _NCHUNK, body, 0)


def _stage_c_body(gs_ref, gd_ref, ef_ref, xx_ref, xy_ref, xz_ref,
                  we2_ref, be2_ref, wc1_ref, bc1_ref, wc2_ref, wr_ref,
                  we_ref, m1_ref, m2_ref):
    B = gs_ref.shape[0]
    p = gs_ref[...] + gd_ref[...]
    x3 = jnp.concatenate([xx_ref[...].reshape(1, B), xy_ref[...].reshape(1, B),
                          xz_ref[...].reshape(1, B)], axis=0)
    eye = (lax.broadcasted_iota(jnp.int32, (B, B), 0)
           == lax.broadcasted_iota(jnp.int32, (B, B), 1)).astype(jnp.float32)
    xd = lax.dot_general(eye, x3, (((1,), (1,)), ((), ())),
                         preferred_element_type=jnp.float32)
    radial = jnp.sum(xd * xd, axis=1, keepdims=True)
    t1 = p + radial * wr_ref[...] + jnp.dot(
        ef_ref[...], we_ref[...], preferred_element_type=jnp.float32)
    u = _silu(t1)
    m = _silu(jnp.dot(u, we2_ref[...], preferred_element_type=jnp.float32)
              + be2_ref[...])
    c2 = _silu(jnp.dot(m, wc1_ref[...], preferred_element_type=jnp.float32)
               + bc1_ref[...])
    coef = jnp.sum(c2 * wc2_ref[...], axis=1, keepdims=True)
    m1_ref[...] = m
    m2_ref[...] = jnp.concatenate(
        [coef * xd, jnp.ones((B, 1), jnp.float32),
         jnp.zeros((B, _D - _CD - 1), jnp.float32)], axis=1)


def _stage_c(gs, gd, ef, xx, xy, xz, We2, be2, Wc1, bc1, wc2row, wr, We):
    B = 512
    full = lambda i: (0, 0)
    return pl.pallas_call(
        _stage_c_body,
        grid=(_E // B,),
        in_specs=[
            pl.BlockSpec((B, _D), lambda i: (i, 0)),
            pl.BlockSpec((B, _D), lambda i: (i, 0)),
            pl.BlockSpec((B, _DE), lambda i: (i, 0)),
            pl.BlockSpec((B,), lambda i: (i,)),
            pl.BlockSpec((B,), lambda i: (i,)),
            pl.BlockSpec((B,), lambda i: (i,)),
            pl.BlockSpec((_H, _H), full),
            pl.BlockSpec((1, _H), full),
            pl.BlockSpec((_H, _H), full),
            pl.BlockSpec((1, _H), full),
            pl.BlockSpec((1, _H), full),
            pl.BlockSpec((1, _H), full),
            pl.BlockSpec((_DE, _H), full),
        ],
        out_specs=[pl.BlockSpec((B, _D), lambda i: (i, 0))] * 2,
        out_shape=[jax.ShapeDtypeStruct((_E, _D), jnp.float32)] * 2,
    )(gs, gd, ef, xx, xy, xz, We2, be2, Wc1, bc1, wc2row, wr, We)


def _make_scatter():
    @functools.partial(
        pl.kernel,
        mesh=_mesh,
        out_type=jax.ShapeDtypeStruct((_NC, _N, _D), jnp.float32),
        scratch_types=[
            pltpu.VMEM((_KB,), jnp.int32),
            pltpu.VMEM((_KB, _D), jnp.float32),
            pltpu.VMEM_SHARED((_N, _D), jnp.float32),
        ],
    )
    def _scatter(m_hbm, dst_hbm, zeros_hbm, out_hbm, idx_v, upd_v, acc_sh):
        cid = lax.axis_index("c")
        sid = lax.axis_index("s")
        @pl.when(sid < 15)
        def _():
            r0 = sid * 640
            pltpu.sync_copy(zeros_hbm.at[pl.ds(r0, 640)],
                            acc_sh.at[pl.ds(r0, 640)])
        @pl.when(sid == 15)
        def _():
            pltpu.sync_copy(zeros_hbm.at[pl.ds(9600, 400)],
                            acc_sh.at[pl.ds(9600, 400)])
        plsc.subcore_barrier()
        base = cid * (_E // _NC) + sid * _EPW

        def body(ci, carry):
            off = base + ci * _KB
            pltpu.sync_copy(dst_hbm.at[pl.ds(off, _KB)], idx_v)
            pltpu.sync_copy(m_hbm.at[pl.ds(off, _KB)], upd_v)
            pltpu.sync_copy(upd_v, acc_sh.at[idx_v], add=True)
            return carry

        lax.fori_loop(0, _NCHUNK, body, 0)
        plsc.subcore_barrier()
        @pl.when(sid < 15)
        def _():
            r0 = sid * 640
            pltpu.sync_copy(acc_sh.at[pl.ds(r0, 640)],
                            out_hbm.at[cid, pl.ds(r0, 640)])
        @pl.when(sid == 15)
        def _():
            pltpu.sync_copy(acc_sh.at[pl.ds(9600, 400)],
                            out_hbm.at[cid, pl.ds(9600, 400)])

    return _scatter


_stage_d = _make_scatter()


def _stage_e_body(h0_ref, h1_ref, x0_ref, x1_ref, node_ref, coord_ref,
                  wn1a_ref, wn1b_ref, bn1_ref, wn2_ref, bn2_ref,
                  h_ref, x_ref):
    hn = h0_ref[...] + h1_ref[...]
    t2 = x0_ref[...] + x1_ref[...]
    xs = t2[:, 0:_CD]
    deg = t2[:, _CD:_CD + 1]
    h1 = _silu(jnp.dot(node_ref[...], wn1a_ref[...],
                       preferred_element_type=jnp.float32)
               + jnp.dot(hn, wn1b_ref[...], preferred_element_type=jnp.float32)
               + bn1_ref[...])
    h_ref[...] = jnp.dot(h1, wn2_ref[...],
                         preferred_element_type=jnp.float32) + bn2_ref[...]
    x_ref[...] = coord_ref[...] + xs / jnp.maximum(deg, 1.0)


def _stage_e(hp0, hp1, xp0, xp1, node, coord, Wn1a, Wn1b, bn1, Wn2, bn2):
    NB = 1000
    full = lambda i: (0, 0)
    return pl.pallas_call(
        _stage_e_body,
        grid=(_N // NB,),
        in_specs=[
            pl.BlockSpec((NB, _D), lambda i: (i, 0)),
            pl.BlockSpec((NB, _D), lambda i: (i, 0)),
            pl.BlockSpec((NB, _D), lambda i: (i, 0)),
            pl.BlockSpec((NB, _D), lambda i: (i, 0)),
            pl.BlockSpec((NB, _D), lambda i: (i, 0)),
            pl.BlockSpec((NB, _CD), lambda i: (i, 0)),
            pl.BlockSpec((_D, _H), full),
            pl.BlockSpec((_H, _H), full),
            pl.BlockSpec((1, _H), full),
            pl.BlockSpec((_H, _D), full),
            pl.BlockSpec((1, _D), full),
        ],
        out_specs=[
            pl.BlockSpec((NB, _D), lambda i: (i, 0)),
            pl.BlockSpec((NB, _CD), lambda i: (i, 0)),
        ],
        out_shape=[
            jax.ShapeDtypeStruct((_N, _D), jnp.float32),
            jax.ShapeDtypeStruct((_N, _CD), jnp.float32),
        ],
    )(hp0, hp1, xp0, xp1, node, coord, Wn1a, Wn1b, bn1, Wn2, bn2)


def kernel(node_feat, coord_feat, edge_feat, edge_index,
           We1, be1, We2, be2, Wn1, bn1, Wn2, bn2, Wc1, bc1, Wc2):
    src = edge_index[0]
    dst = edge_index[1]
    Ws = We1[0:_D]
    Wd = We1[_D:2 * _D]
    wr = We1[2 * _D:2 * _D + 1]
    We = We1[2 * _D + 1:]
    cx = coord_feat[:, 0]
    cy = coord_feat[:, 1]
    cz = coord_feat[:, 2]
    ts, td = _stage_a(node_feat, Ws, Wd, be1.reshape(1, _D))
    gs, gd, xx, xy, xz = _stage_b(ts, td, src, dst, cx, cy, cz)
    m1, m2 = _stage_c(gs, gd, edge_feat, xx, xy, xz, We2,
                      be2.reshape(1, _H), Wc1, bc1.reshape(1, _H),
                      Wc2.reshape(1, _H), wr, We)
    zeros = jnp.zeros((_N, _D), jnp.float32)
    p1 = _stage_d(m1, dst, zeros)
    p2 = _stage_d(m2, dst, zeros)
    h, x = _stage_e(p1[0], p1[1], p2[0], p2[1], node_feat, coord_feat,
                    Wn1[0:_D], Wn1[_D:], bn1.reshape(1, _H), Wn2,
                    bn2.reshape(1, _D))
    return (h, x)
